# baseline (speedup 1.0000x reference)
import jax
import jax.numpy as jnp
from jax import lax
from jax.experimental import pallas as pl
from jax.experimental.pallas import tpu as pltpu, tpu_sc as plsc

N = 10000
E = 320000
DN = 128
H2 = 128

NC, NS = 2, 16
NW = NC * NS
EPW = E // NW
C1 = 80
G1 = EPW // C1
C3 = 400
G3 = EPW // C3
NP = 10240
RPT = NP // NS

_mesh = plsc.VectorSubcoreMesh(core_axis_name="c", subcore_axis_name="s")
F32 = jnp.float32


def _dot(a, b):
    return jnp.dot(a, b, preferred_element_type=F32)


def _k_tables(x_ref, wr_ref, ws_ref, tr_ref, ts_ref):
    xb = x_ref[...]
    tr_ref[...] = _dot(xb, wr_ref[...])
    ts_ref[...] = _dot(xb, ws_ref[...])


def _k_q(ea_ref, we_ref, u_ref, wu_ref, b_ref, q_ref):
    q_ref[...] = _dot(ea_ref[...], we_ref[...]) + _dot(u_ref[...], wu_ref[...]) + b_ref[...]


def _sc1_body(tr_hbm, ts_hbm, q_hbm, r_hbm, s_hbm,
              e1_hbm, agg_hbm,
              ir_v, is_v, tr_v, ts_v, q_v, e1_v, agg_sh,
              sem_r, sem_s, sem_q, sem_st):
    cid = lax.axis_index("c")
    sid = lax.axis_index("s")
    wid = sid * NC + cid

    zvec = jnp.zeros((16,), F32)

    def zfill(t, carry):
        e1_v[t // 8, pl.ds((t % 8) * 16, 16)] = zvec
        return carry
    lax.fori_loop(0, C1 * 8, zfill, None)

    def zcopy(k, carry):
        off = pl.multiple_of(sid * RPT + k * C1, C1)
        pltpu.sync_copy(e1_v, agg_sh.at[pl.ds(off, C1)])
        return carry
    lax.fori_loop(0, RPT // C1, zcopy, None)
    plsc.subcore_barrier()

    def chunk(g, carry):
        base = pl.multiple_of(wid * EPW + g * C1, C1)
        pltpu.sync_copy(r_hbm.at[pl.ds(base, C1)], ir_v)
        pltpu.sync_copy(s_hbm.at[pl.ds(base, C1)], is_v)
        cp_r = pltpu.async_copy(tr_hbm.at[ir_v], tr_v, sem_r)
        cp_s = pltpu.async_copy(ts_hbm.at[is_v], ts_v, sem_s)
        cp_q = pltpu.async_copy(q_hbm.at[pl.ds(base, C1)], q_v, sem_q)
        cp_r.wait()
        cp_s.wait()
        cp_q.wait()

        @plsc.parallel_loop(0, C1, 1, unroll=4)
        def row(i):
            for j in range(H2 // 16):
                sl = pl.ds(j * 16, 16)
                v = q_v[i, sl] + tr_v[i, sl] + ts_v[i, sl]
                e1_v[i, sl] = jnp.maximum(v, 0.0)

        pltpu.sync_copy(e1_v, e1_hbm.at[pl.ds(base, C1)])
        pltpu.sync_copy(e1_v, agg_sh.at[ir_v], add=True)
        return carry
    lax.fori_loop(0, G1, chunk, None)

    plsc.subcore_barrier()
    doff = pl.multiple_of(sid * RPT, RPT)
    pltpu.sync_copy(agg_sh.at[pl.ds(doff, RPT)],
                    agg_hbm.at[cid, pl.ds(doff, RPT)])


def _k_node(agg0_ref, agg1_ref, x_ref, u_ref, wna_ref, wnx_ref, wnu_ref, nb_ref,
            wdr_ref, wds_ref, tbr_ref, tbs_ref, csa_ref, csn_ref):
    i = pl.program_id(0)
    agg = agg0_ref[...] + agg1_ref[...]
    n1 = jnp.maximum(_dot(agg, wna_ref[...]) + _dot(x_ref[...], wnx_ref[...])
                     + _dot(u_ref[...], wnu_ref[...]) + nb_ref[...], 0.0)
    tbr_ref[...] = _dot(n1, wdr_ref[...])
    tbs_ref[...] = _dot(n1, wds_ref[...])

    @pl.when(i == 0)
    def _():
        csa_ref[...] = jnp.zeros_like(csa_ref)
        csn_ref[...] = jnp.zeros_like(csn_ref)
    csa_ref[...] += jnp.sum(agg, axis=0, keepdims=True)
    csn_ref[...] += jnp.sum(n1, axis=0, keepdims=True)


def _k_c2(csa_ref, csn_ref, u_ref, gbw_ref, gbb_ref, wdu_ref, db1_ref, c2_ref):
    u = u_ref[...]
    for t in range(2):
        sl = slice(t * 64, (t + 1) * 64)
        me1 = csa_ref[:, sl] * (1.0 / E)
        mn1 = csn_ref[:, sl] * (1.0 / N)
        g = jnp.concatenate([me1, mn1, u], axis=1)
        u1 = jnp.maximum(_dot(g, gbw_ref[t]) + gbb_ref[t], 0.0)
        c2_ref[:, sl] = db1_ref[t] + _dot(u1, wdu_ref[t])


def _sc3_body(tbr_hbm, tbs_hbm, r_hbm, s_hbm, x2_hbm,
              gsum_hbm, xr2_hbm, xs2_hbm,
              ir_v, is_v, tbr_v, tbs_v, xr2_v, xs2_v,
              sem_r, sem_s, sem_x1, sem_x2, sem_st):
    cid = lax.axis_index("c")
    sid = lax.axis_index("s")
    wid = sid * NC + cid

    def chunk(g, carry):
        base = pl.multiple_of(wid * EPW + g * C3, C3)
        pltpu.sync_copy(r_hbm.at[pl.ds(base, C3)], ir_v)
        pltpu.sync_copy(s_hbm.at[pl.ds(base, C3)], is_v)
        cp_r = pltpu.async_copy(tbr_hbm.at[ir_v], tbr_v, sem_r)
        cp_s = pltpu.async_copy(tbs_hbm.at[is_v], tbs_v, sem_s)
        cp_x1 = pltpu.async_copy(x2_hbm.at[ir_v], xr2_v, sem_x1)
        cp_x2 = pltpu.async_copy(x2_hbm.at[is_v], xs2_v, sem_x2)
        cp_r.wait()
        cp_s.wait()

        @plsc.parallel_loop(0, C3, 1, unroll=4)
        def row(i):
            for j in range(H2 // 16):
                sl = pl.ds(j * 16, 16)
                tbr_v[i, sl] = tbr_v[i, sl] + tbs_v[i, sl]

        pltpu.sync_copy(tbr_v, gsum_hbm.at[pl.ds(base, C3)])
        cp_x1.wait()
        cp_x2.wait()
        pltpu.sync_copy(xr2_v, xr2_hbm.at[pl.ds(base, C3)])
        pltpu.sync_copy(xs2_v, xs2_hbm.at[pl.ds(base, C3)])
        return carry
    lax.fori_loop(0, G3, chunk, None)


def _k_final(e1_ref, gs_ref, xr2_ref, xs2_ref, c2_ref, wde_ref, w2_ref, b2_ref,
             out_ref):
    h = jnp.maximum(_dot(e1_ref[...], wde_ref[...]) + gs_ref[...] + c2_ref[...], 0.0)
    o = _dot(h, w2_ref[...]) + b2_ref[...]
    out_ref[...] = o[:, 0:1] * (xr2_ref[...] - o[:, 1:2] * xs2_ref[...])


def _cat(a, b):
    return jnp.concatenate([a, b], axis=1)


def _bd(a, b):
    z = jnp.zeros_like(a)
    return jnp.concatenate([jnp.concatenate([a, z], 1), jnp.concatenate([z, b], 1)], 0)


@jax.jit
def kernel(x, edge_attr, global_attr, params, edge_index):
    p1, p2 = params["w1"], params["w2"]
    u = global_attr

    We = _cat(p1["eb_W"][0:16], p2["eb_W"][0:16])
    Wr = _cat(p1["eb_W"][16:144], p2["eb_W"][16:144])
    Ws = _cat(p1["eb_W"][144:272], p2["eb_W"][144:272])
    Wu = _cat(p1["eb_W"][272:336], p2["eb_W"][272:336])
    eb_b = _cat(p1["eb_b"][None], p2["eb_b"][None])
    Wna = _bd(p1["nb_W"][0:64], p2["nb_W"][0:64])
    Wnx = _cat(p1["nb_W"][64:192], p2["nb_W"][64:192])
    Wnu = _cat(p1["nb_W"][192:256], p2["nb_W"][192:256])
    nb_b = _cat(p1["nb_b"][None], p2["nb_b"][None])
    gbW = jnp.stack([p1["gb_W"], p2["gb_W"]])
    gbb = jnp.stack([p1["gb_b"][None], p2["gb_b"][None]])
    Wde = _bd(p1["dec_W1"][0:64], p2["dec_W1"][0:64])
    Wdr = _bd(p1["dec_W1"][64:128], p2["dec_W1"][64:128])
    Wds = _bd(p1["dec_W1"][128:192], p2["dec_W1"][128:192])
    Wdu = jnp.stack([p1["dec_W1"][192:256], p2["dec_W1"][192:256]])
    db1 = jnp.stack([p1["dec_b1"][None], p2["dec_b1"][None]])
    z64 = jnp.zeros((64, 1), F32)
    W2 = jnp.concatenate([jnp.concatenate([p1["dec_W2"], z64], 0),
                          jnp.concatenate([z64, p2["dec_W2"]], 0),
                          jnp.zeros((128, 6), F32)], axis=1)
    b2 = jnp.concatenate([p1["dec_b2"], p2["dec_b2"], jnp.zeros((6,), F32)])[None]

    s_idx = edge_index[0]
    r_idx = edge_index[1]

    BN = 1000
    BE = 2000

    def full(shp):
        return pl.BlockSpec(shp, lambda *a: tuple(0 for _ in shp))

    tr_tab, ts_tab = pl.pallas_call(
        _k_tables,
        grid=(N // BN,),
        in_specs=[pl.BlockSpec((BN, DN), lambda i: (i, 0)), full((DN, H2)), full((DN, H2))],
        out_specs=[pl.BlockSpec((BN, H2), lambda i: (i, 0))] * 2,
        out_shape=[jax.ShapeDtypeStruct((N, H2), F32)] * 2,
    )(x, Wr, Ws)

    q = pl.pallas_call(
        _k_q,
        grid=(E // BE,),
        in_specs=[pl.BlockSpec((BE, 16), lambda i: (i, 0)), full((16, H2)),
                  full((1, 64)), full((64, H2)), full((1, H2))],
        out_specs=pl.BlockSpec((BE, H2), lambda i: (i, 0)),
        out_shape=jax.ShapeDtypeStruct((E, H2), F32),
    )(edge_attr, We, u, Wu, eb_b)

    sc1 = pl.kernel(
        _sc1_body,
        out_type=(jax.ShapeDtypeStruct((E, H2), F32),
                  jax.ShapeDtypeStruct((NC, NP, H2), F32)),
        mesh=_mesh,
        scratch_types=(
            pltpu.VMEM((C1,), jnp.int32),
            pltpu.VMEM((C1,), jnp.int32),
            pltpu.VMEM((C1, H2), F32),
            pltpu.VMEM((C1, H2), F32),
            pltpu.VMEM((C1, H2), F32),
            pltpu.VMEM((C1, H2), F32),
            pltpu.VMEM_SHARED((NP, H2), F32),
            pltpu.SemaphoreType.DMA,
            pltpu.SemaphoreType.DMA,
            pltpu.SemaphoreType.DMA,
            pltpu.SemaphoreType.DMA,
        ),
    )
    e1, aggp = sc1(tr_tab, ts_tab, q, r_idx, s_idx)

    tbr, tbs, csa, csn = pl.pallas_call(
        _k_node,
        grid=(N // BN,),
        in_specs=[pl.BlockSpec((BN, H2), lambda i: (i, 0)),
                  pl.BlockSpec((BN, H2), lambda i: (i, 0)),
                  pl.BlockSpec((BN, DN), lambda i: (i, 0)),
                  full((1, 64)), full((DN, H2)), full((DN, H2)), full((64, H2)),
                  full((1, H2)), full((DN, H2)), full((DN, H2))],
        out_specs=[pl.BlockSpec((BN, H2), lambda i: (i, 0)),
                   pl.BlockSpec((BN, H2), lambda i: (i, 0)),
                   full((1, H2)), full((1, H2))],
        out_shape=[jax.ShapeDtypeStruct((N, H2), F32),
                   jax.ShapeDtypeStruct((N, H2), F32),
                   jax.ShapeDtypeStruct((1, H2), F32),
                   jax.ShapeDtypeStruct((1, H2), F32)],
    )(aggp[0, :N], aggp[1, :N], x, u, Wna, Wnx, Wnu, nb_b, Wdr, Wds)

    c2 = pl.pallas_call(
        _k_c2,
        in_specs=[full((1, H2)), full((1, H2)), full((1, 64)),
                  full((2, 192, 64)), full((2, 1, 64)), full((2, 64, 64)),
                  full((2, 1, 64))],
        out_specs=full((1, H2)),
        out_shape=jax.ShapeDtypeStruct((1, H2), F32),
    )(csa, csn, u, gbW, gbb, Wdu, db1)

    sc3 = pl.kernel(
        _sc3_body,
        out_type=(jax.ShapeDtypeStruct((E, H2), F32),
                  jax.ShapeDtypeStruct((E,), F32),
                  jax.ShapeDtypeStruct((E,), F32)),
        mesh=_mesh,
        scratch_types=(
            pltpu.VMEM((C3,), jnp.int32),
            pltpu.VMEM((C3,), jnp.int32),
            pltpu.VMEM((C3, H2), F32),
            pltpu.VMEM((C3, H2), F32),
            pltpu.VMEM((C3,), F32),
            pltpu.VMEM((C3,), F32),
            pltpu.SemaphoreType.DMA,
            pltpu.SemaphoreType.DMA,
            pltpu.SemaphoreType.DMA,
            pltpu.SemaphoreType.DMA,
            pltpu.SemaphoreType.DMA,
        ),
    )
    x2col = x[:, 2]
    gsum, xr2, xs2 = sc3(tbr, tbs, r_idx, s_idx, x2col)

    out = pl.pallas_call(
        _k_final,
        grid=(E // BE,),
        in_specs=[pl.BlockSpec((BE, H2), lambda i: (i, 0)),
                  pl.BlockSpec((BE, H2), lambda i: (i, 0)),
                  pl.BlockSpec((BE, 1), lambda i: (i, 0)),
                  pl.BlockSpec((BE, 1), lambda i: (i, 0)),
                  full((1, H2)), full((DN, H2)), full((DN, 8)), full((1, 8))],
        out_specs=pl.BlockSpec((BE, 1), lambda i: (i, 0)),
        out_shape=jax.ShapeDtypeStruct((E, 1), F32),
    )(e1, gsum, xr2[:, None], xs2[:, None], c2, Wde, W2, b2)

    return out


# speedup vs baseline: 1.1371x; 1.1371x over previous
import jax
import jax.numpy as jnp
from jax import lax
from jax.experimental import pallas as pl
from jax.experimental.pallas import tpu as pltpu, tpu_sc as plsc

N = 10000
E = 320000
DN = 128
H2 = 128

NC, NS = 2, 16
NW = NC * NS
EPW = E // NW
C1 = 80
G1 = EPW // C1
C3 = 200
G3 = EPW // C3
NP = 10240
RPT = NP // NS

_mesh = plsc.VectorSubcoreMesh(core_axis_name="c", subcore_axis_name="s")
F32 = jnp.float32


def _dot(a, b):
    return jnp.dot(a, b, preferred_element_type=F32)


def _k_tables(x_ref, wr_ref, ws_ref, tr_ref, ts_ref):
    xb = x_ref[...]
    tr_ref[...] = _dot(xb, wr_ref[...])
    ts_ref[...] = _dot(xb, ws_ref[...])


def _k_q(ea_ref, we_ref, u_ref, wu_ref, b_ref, q_ref):
    q_ref[...] = _dot(ea_ref[...], we_ref[...]) + _dot(u_ref[...], wu_ref[...]) + b_ref[...]


def _sc1_body(tr_hbm, ts_hbm, q_hbm, r_hbm, s_hbm, x2_hbm,
              e1_hbm, xr2_hbm, xs2_hbm, agg_hbm,
              ir_v, is_v, tr_v, ts_v, q_v, e1_v, xr2_v, xs2_v, agg_sh,
              sem_r, sem_s, sem_q, sem_x1, sem_x2, sem_st):
    cid = lax.axis_index("c")
    sid = lax.axis_index("s")
    wid = sid * NC + cid

    zvec = jnp.zeros((16,), F32)

    def zfill(t, carry):
        e1_v[t // 8, pl.ds((t % 8) * 16, 16)] = zvec
        return carry
    lax.fori_loop(0, C1 * 8, zfill, None)

    def zcopy(k, carry):
        off = pl.multiple_of(sid * RPT + k * C1, C1)
        pltpu.sync_copy(e1_v, agg_sh.at[pl.ds(off, C1)])
        return carry
    lax.fori_loop(0, RPT // C1, zcopy, None)
    plsc.subcore_barrier()

    def chunk(g, carry):
        base = pl.multiple_of(wid * EPW + g * C1, C1)
        pltpu.sync_copy(r_hbm.at[pl.ds(base, C1)], ir_v)
        pltpu.sync_copy(s_hbm.at[pl.ds(base, C1)], is_v)
        cp_r = pltpu.async_copy(tr_hbm.at[ir_v], tr_v, sem_r)
        cp_s = pltpu.async_copy(ts_hbm.at[is_v], ts_v, sem_s)
        cp_q = pltpu.async_copy(q_hbm.at[pl.ds(base, C1)], q_v, sem_q)
        cp_x1 = pltpu.async_copy(x2_hbm.at[ir_v], xr2_v, sem_x1)
        cp_x2 = pltpu.async_copy(x2_hbm.at[is_v], xs2_v, sem_x2)
        cp_r.wait()
        cp_s.wait()
        cp_q.wait()

        @plsc.parallel_loop(0, C1, 1, unroll=4)
        def row(i):
            for j in range(H2 // 16):
                sl = pl.ds(j * 16, 16)
                v = q_v[i, sl] + tr_v[i, sl] + ts_v[i, sl]
                e1_v[i, sl] = jnp.maximum(v, 0.0)

        cp_st = pltpu.async_copy(e1_v, e1_hbm.at[pl.ds(base, C1)], sem_st)
        pltpu.sync_copy(e1_v, agg_sh.at[ir_v], add=True)
        cp_x1.wait()
        cp_x2.wait()
        pltpu.sync_copy(xr2_v, xr2_hbm.at[pl.ds(base, C1)])
        pltpu.sync_copy(xs2_v, xs2_hbm.at[pl.ds(base, C1)])
        cp_st.wait()
        return carry
    lax.fori_loop(0, G1, chunk, None)

    plsc.subcore_barrier()
    doff = pl.multiple_of(sid * RPT, RPT)
    pltpu.sync_copy(agg_sh.at[pl.ds(doff, RPT)],
                    agg_hbm.at[cid, pl.ds(doff, RPT)])


def _k_node(agg0_ref, agg1_ref, x_ref, u_ref, wna_ref, wnx_ref, wnu_ref, nb_ref,
            wdr_ref, wds_ref, tbr_ref, tbs_ref, csa_ref, csn_ref):
    i = pl.program_id(0)
    agg = agg0_ref[...] + agg1_ref[...]
    n1 = jnp.maximum(_dot(agg, wna_ref[...]) + _dot(x_ref[...], wnx_ref[...])
                     + _dot(u_ref[...], wnu_ref[...]) + nb_ref[...], 0.0)
    tbr_ref[...] = _dot(n1, wdr_ref[...])
    tbs_ref[...] = _dot(n1, wds_ref[...])

    @pl.when(i == 0)
    def _():
        csa_ref[...] = jnp.zeros_like(csa_ref)
        csn_ref[...] = jnp.zeros_like(csn_ref)
    csa_ref[...] += jnp.sum(agg, axis=0, keepdims=True)
    csn_ref[...] += jnp.sum(n1, axis=0, keepdims=True)


def _k_c2(csa_ref, csn_ref, u_ref, gbw_ref, gbb_ref, wdu_ref, db1_ref, c2_ref):
    u = u_ref[...]
    for t in range(2):
        sl = slice(t * 64, (t + 1) * 64)
        me1 = csa_ref[:, sl] * (1.0 / E)
        mn1 = csn_ref[:, sl] * (1.0 / N)
        g = jnp.concatenate([me1, mn1, u], axis=1)
        u1 = jnp.maximum(_dot(g, gbw_ref[t]) + gbb_ref[t], 0.0)
        c2_ref[:, sl] = db1_ref[t] + _dot(u1, wdu_ref[t])


def _sc3_body(tbr_hbm, tbs_hbm, r_hbm, s_hbm, gsum_hbm,
              ir_v, is_v, tbr_v, tbs_v, sem_r, sem_s, sem_st):
    cid = lax.axis_index("c")
    sid = lax.axis_index("s")
    wid = sid * NC + cid

    def chunk(g, carry):
        base = pl.multiple_of(wid * EPW + g * C3, C3)
        pltpu.sync_copy(r_hbm.at[pl.ds(base, C3)], ir_v)
        pltpu.sync_copy(s_hbm.at[pl.ds(base, C3)], is_v)
        cp_r = pltpu.async_copy(tbr_hbm.at[ir_v], tbr_v, sem_r)
        cp_s = pltpu.async_copy(tbs_hbm.at[is_v], tbs_v, sem_s)
        cp_r.wait()
        cp_s.wait()

        @plsc.parallel_loop(0, C3, 1, unroll=4)
        def row(i):
            for j in range(H2 // 16):
                sl = pl.ds(j * 16, 16)
                tbr_v[i, sl] = tbr_v[i, sl] + tbs_v[i, sl]

        pltpu.sync_copy(tbr_v, gsum_hbm.at[pl.ds(base, C3)])
        return carry
    lax.fori_loop(0, G3, chunk, None)


def _k_final(e1_ref, gs_ref, xr2_ref, xs2_ref, c2_ref, wde_ref, w2_ref, b2_ref,
             out_ref):
    h = jnp.maximum(_dot(e1_ref[...], wde_ref[...]) + gs_ref[...] + c2_ref[...], 0.0)
    o = _dot(h, w2_ref[...]) + b2_ref[...]
    out_ref[...] = o[:, 0:1] * (xr2_ref[...] - o[:, 1:2] * xs2_ref[...])


def _cat(a, b):
    return jnp.concatenate([a, b], axis=1)


def _bd(a, b):
    z = jnp.zeros_like(a)
    return jnp.concatenate([jnp.concatenate([a, z], 1), jnp.concatenate([z, b], 1)], 0)


@jax.jit
def kernel(x, edge_attr, global_attr, params, edge_index):
    p1, p2 = params["w1"], params["w2"]
    u = global_attr

    We = _cat(p1["eb_W"][0:16], p2["eb_W"][0:16])
    Wr = _cat(p1["eb_W"][16:144], p2["eb_W"][16:144])
    Ws = _cat(p1["eb_W"][144:272], p2["eb_W"][144:272])
    Wu = _cat(p1["eb_W"][272:336], p2["eb_W"][272:336])
    eb_b = _cat(p1["eb_b"][None], p2["eb_b"][None])
    Wna = _bd(p1["nb_W"][0:64], p2["nb_W"][0:64])
    Wnx = _cat(p1["nb_W"][64:192], p2["nb_W"][64:192])
    Wnu = _cat(p1["nb_W"][192:256], p2["nb_W"][192:256])
    nb_b = _cat(p1["nb_b"][None], p2["nb_b"][None])
    gbW = jnp.stack([p1["gb_W"], p2["gb_W"]])
    gbb = jnp.stack([p1["gb_b"][None], p2["gb_b"][None]])
    Wde = _bd(p1["dec_W1"][0:64], p2["dec_W1"][0:64])
    Wdr = _bd(p1["dec_W1"][64:128], p2["dec_W1"][64:128])
    Wds = _bd(p1["dec_W1"][128:192], p2["dec_W1"][128:192])
    Wdu = jnp.stack([p1["dec_W1"][192:256], p2["dec_W1"][192:256]])
    db1 = jnp.stack([p1["dec_b1"][None], p2["dec_b1"][None]])
    z64 = jnp.zeros((64, 1), F32)
    W2 = jnp.concatenate([jnp.concatenate([p1["dec_W2"], z64], 0),
                          jnp.concatenate([z64, p2["dec_W2"]], 0),
                          jnp.zeros((128, 6), F32)], axis=1)
    b2 = jnp.concatenate([p1["dec_b2"], p2["dec_b2"], jnp.zeros((6,), F32)])[None]

    s_idx = edge_index[0]
    r_idx = edge_index[1]

    BN = 1000
    BE = 2000

    def full(shp):
        return pl.BlockSpec(shp, lambda *a: tuple(0 for _ in shp))

    tr_tab, ts_tab = pl.pallas_call(
        _k_tables,
        grid=(N // BN,),
        in_specs=[pl.BlockSpec((BN, DN), lambda i: (i, 0)), full((DN, H2)), full((DN, H2))],
        out_specs=[pl.BlockSpec((BN, H2), lambda i: (i, 0))] * 2,
        out_shape=[jax.ShapeDtypeStruct((N, H2), F32)] * 2,
    )(x, Wr, Ws)

    q = pl.pallas_call(
        _k_q,
        grid=(E // BE,),
        in_specs=[pl.BlockSpec((BE, 16), lambda i: (i, 0)), full((16, H2)),
                  full((1, 64)), full((64, H2)), full((1, H2))],
        out_specs=pl.BlockSpec((BE, H2), lambda i: (i, 0)),
        out_shape=jax.ShapeDtypeStruct((E, H2), F32),
    )(edge_attr, We, u, Wu, eb_b)

    sc1 = pl.kernel(
        _sc1_body,
        out_type=(jax.ShapeDtypeStruct((E, H2), F32),
                  jax.ShapeDtypeStruct((E,), F32),
                  jax.ShapeDtypeStruct((E,), F32),
                  jax.ShapeDtypeStruct((NC, NP, H2), F32)),
        mesh=_mesh,
        scratch_types=(
            pltpu.VMEM((C1,), jnp.int32),
            pltpu.VMEM((C1,), jnp.int32),
            pltpu.VMEM((C1, H2), F32),
            pltpu.VMEM((C1, H2), F32),
            pltpu.VMEM((C1, H2), F32),
            pltpu.VMEM((C1, H2), F32),
            pltpu.VMEM((C1,), F32),
            pltpu.VMEM((C1,), F32),
            pltpu.VMEM_SHARED((NP, H2), F32),
            pltpu.SemaphoreType.DMA,
            pltpu.SemaphoreType.DMA,
            pltpu.SemaphoreType.DMA,
            pltpu.SemaphoreType.DMA,
            pltpu.SemaphoreType.DMA,
            pltpu.SemaphoreType.DMA,
        ),
    )
    x2col = x[:, 2]
    e1, xr2, xs2, aggp = sc1(tr_tab, ts_tab, q, r_idx, s_idx, x2col)

    tbr, tbs, csa, csn = pl.pallas_call(
        _k_node,
        grid=(N // BN,),
        in_specs=[pl.BlockSpec((BN, H2), lambda i: (i, 0)),
                  pl.BlockSpec((BN, H2), lambda i: (i, 0)),
                  pl.BlockSpec((BN, DN), lambda i: (i, 0)),
                  full((1, 64)), full((DN, H2)), full((DN, H2)), full((64, H2)),
                  full((1, H2)), full((DN, H2)), full((DN, H2))],
        out_specs=[pl.BlockSpec((BN, H2), lambda i: (i, 0)),
                   pl.BlockSpec((BN, H2), lambda i: (i, 0)),
                   full((1, H2)), full((1, H2))],
        out_shape=[jax.ShapeDtypeStruct((N, H2), F32),
                   jax.ShapeDtypeStruct((N, H2), F32),
                   jax.ShapeDtypeStruct((1, H2), F32),
                   jax.ShapeDtypeStruct((1, H2), F32)],
    )(aggp[0, :N], aggp[1, :N], x, u, Wna, Wnx, Wnu, nb_b, Wdr, Wds)

    c2 = pl.pallas_call(
        _k_c2,
        in_specs=[full((1, H2)), full((1, H2)), full((1, 64)),
                  full((2, 192, 64)), full((2, 1, 64)), full((2, 64, 64)),
                  full((2, 1, 64))],
        out_specs=full((1, H2)),
        out_shape=jax.ShapeDtypeStruct((1, H2), F32),
    )(csa, csn, u, gbW, gbb, Wdu, db1)

    sc3 = pl.kernel(
        _sc3_body,
        out_type=jax.ShapeDtypeStruct((E, H2), F32),
        mesh=_mesh,
        scratch_types=(
            pltpu.VMEM((C3,), jnp.int32),
            pltpu.VMEM((C3,), jnp.int32),
            pltpu.VMEM((C3, H2), F32),
            pltpu.VMEM((C3, H2), F32),
            pltpu.SemaphoreType.DMA,
            pltpu.SemaphoreType.DMA,
            pltpu.SemaphoreType.DMA,
        ),
    )
    gsum = sc3(tbr, tbs, r_idx, s_idx)

    out = pl.pallas_call(
        _k_final,
        grid=(E // BE,),
        in_specs=[pl.BlockSpec((BE, H2), lambda i: (i, 0)),
                  pl.BlockSpec((BE, H2), lambda i: (i, 0)),
                  pl.BlockSpec((BE, 1), lambda i: (i, 0)),
                  pl.BlockSpec((BE, 1), lambda i: (i, 0)),
                  full((1, H2)), full((DN, H2)), full((DN, 8)), full((1, 8))],
        out_specs=pl.BlockSpec((BE, 1), lambda i: (i, 0)),
        out_shape=jax.ShapeDtypeStruct((E, 1), F32),
    )(e1, gsum, xr2[:, None], xs2[:, None], c2, Wde, W2, b2)

    return out


# speedup vs baseline: 1.2608x; 1.1088x over previous
import jax
import jax.numpy as jnp
from jax import lax
from jax.experimental import pallas as pl
from jax.experimental.pallas import tpu as pltpu, tpu_sc as plsc

N = 10000
E = 320000
DN = 128
H2 = 128

NC, NS = 2, 16
NW = NC * NS
EPW = E // NW
C1 = 80
G1 = EPW // C1
C3 = 200
G3 = EPW // C3
NP = 10240
RPT = NP // NS

_mesh = plsc.VectorSubcoreMesh(core_axis_name="c", subcore_axis_name="s")
F32 = jnp.float32


def _dot(a, b):
    return jnp.dot(a, b, preferred_element_type=F32)


def _k_tables(x_ref, wr_ref, ws_ref, tr_ref, ts_ref):
    xb = x_ref[...]
    tr_ref[...] = _dot(xb, wr_ref[...])
    ts_ref[...] = _dot(xb, ws_ref[...])


def _k_q(ea_ref, we_ref, u_ref, wu_ref, b_ref, q_ref):
    q_ref[...] = _dot(ea_ref[...], we_ref[...]) + _dot(u_ref[...], wu_ref[...]) + b_ref[...]


def _sc1_body(tr_hbm, ts_hbm, q_hbm, r_hbm, s_hbm, x2_hbm,
              e1_hbm, xr2_hbm, xs2_hbm, agg_hbm,
              ir_v, is_v, tr_v, ts_v, q_v, e1_v, xr2_v, xs2_v, agg_sh,
              sem_r, sem_s, sem_q, sem_x1, sem_x2, sem_st):
    cid = lax.axis_index("c")
    sid = lax.axis_index("s")
    wid = sid * NC + cid

    zvec = jnp.zeros((16,), F32)

    def zfill(t, carry):
        e1_v[t // 8, pl.ds((t % 8) * 16, 16)] = zvec
        return carry
    lax.fori_loop(0, C1 * 8, zfill, None)

    def zcopy(k, carry):
        off = pl.multiple_of(sid * RPT + k * C1, C1)
        pltpu.sync_copy(e1_v, agg_sh.at[pl.ds(off, C1)])
        return carry
    lax.fori_loop(0, RPT // C1, zcopy, None)
    plsc.subcore_barrier()

    def chunk(g, carry):
        base = pl.multiple_of(wid * EPW + g * C1, C1)
        pltpu.sync_copy(r_hbm.at[pl.ds(base, C1)], ir_v)
        pltpu.sync_copy(s_hbm.at[pl.ds(base, C1)], is_v)
        cp_r = pltpu.async_copy(tr_hbm.at[ir_v], tr_v, sem_r)
        cp_s = pltpu.async_copy(ts_hbm.at[is_v], ts_v, sem_s)
        cp_q = pltpu.async_copy(q_hbm.at[pl.ds(base, C1)], q_v, sem_q)
        cp_x1 = pltpu.async_copy(x2_hbm.at[ir_v], xr2_v, sem_x1)
        cp_x2 = pltpu.async_copy(x2_hbm.at[is_v], xs2_v, sem_x2)
        cp_r.wait()
        cp_s.wait()
        cp_q.wait()

        @plsc.parallel_loop(0, C1, 1, unroll=4)
        def row(i):
            for j in range(H2 // 16):
                sl = pl.ds(j * 16, 16)
                v = q_v[i, sl] + tr_v[i, sl] + ts_v[i, sl]
                e1_v[i, sl] = jnp.maximum(v, 0.0)

        cp_st = pltpu.async_copy(e1_v, e1_hbm.at[pl.ds(base, C1)], sem_st)
        pltpu.sync_copy(e1_v, agg_sh.at[ir_v], add=True)
        cp_x1.wait()
        cp_x2.wait()
        pltpu.sync_copy(xr2_v, xr2_hbm.at[pl.ds(base, C1)])
        pltpu.sync_copy(xs2_v, xs2_hbm.at[pl.ds(base, C1)])
        cp_st.wait()
        return carry
    lax.fori_loop(0, G1, chunk, None)

    plsc.subcore_barrier()
    doff = pl.multiple_of(sid * RPT, RPT)
    pltpu.sync_copy(agg_sh.at[pl.ds(doff, RPT)],
                    agg_hbm.at[cid, pl.ds(doff, RPT)])


def _k_node(agg0_ref, agg1_ref, x_ref, u_ref, wna_ref, wnx_ref, wnu_ref, nb_ref,
            wdr_ref, wds_ref, tbr_ref, tbs_ref, csa_ref, csn_ref):
    i = pl.program_id(0)
    agg = agg0_ref[...] + agg1_ref[...]
    n1 = jnp.maximum(_dot(agg, wna_ref[...]) + _dot(x_ref[...], wnx_ref[...])
                     + _dot(u_ref[...], wnu_ref[...]) + nb_ref[...], 0.0)
    tbr_ref[...] = _dot(n1, wdr_ref[...])
    tbs_ref[...] = _dot(n1, wds_ref[...])

    @pl.when(i == 0)
    def _():
        csa_ref[...] = jnp.zeros_like(csa_ref)
        csn_ref[...] = jnp.zeros_like(csn_ref)
    csa_ref[...] += jnp.sum(agg, axis=0, keepdims=True)
    csn_ref[...] += jnp.sum(n1, axis=0, keepdims=True)


def _k_c2(csa_ref, csn_ref, u_ref, gbw_ref, gbb_ref, wdu_ref, db1_ref, c2_ref):
    u = u_ref[...]
    for t in range(2):
        sl = slice(t * 64, (t + 1) * 64)
        me1 = csa_ref[:, sl] * (1.0 / E)
        mn1 = csn_ref[:, sl] * (1.0 / N)
        g = jnp.concatenate([me1, mn1, u], axis=1)
        u1 = jnp.maximum(_dot(g, gbw_ref[t]) + gbb_ref[t], 0.0)
        c2_ref[:, sl] = db1_ref[t] + _dot(u1, wdu_ref[t])


def _sc3_body(tbr_hbm, tbs_hbm, r_hbm, s_hbm, gsum_hbm,
              ir_v, is_v, tbr_v, tbs_v, sem_r, sem_s, sem_st):
    cid = lax.axis_index("c")
    sid = lax.axis_index("s")
    wid = sid * NC + cid

    def chunk(g, carry):
        base = pl.multiple_of(wid * EPW + g * C3, C3)
        pltpu.sync_copy(r_hbm.at[pl.ds(base, C3)], ir_v)
        pltpu.sync_copy(s_hbm.at[pl.ds(base, C3)], is_v)
        cp_r = pltpu.async_copy(tbr_hbm.at[ir_v], tbr_v, sem_r)
        cp_s = pltpu.async_copy(tbs_hbm.at[is_v], tbs_v, sem_s)
        cp_r.wait()
        cp_s.wait()

        @plsc.parallel_loop(0, C3, 1, unroll=4)
        def row(i):
            for j in range(H2 // 16):
                sl = pl.ds(j * 16, 16)
                tbr_v[i, sl] = tbr_v[i, sl] + tbs_v[i, sl]

        pltpu.sync_copy(tbr_v, gsum_hbm.at[pl.ds(base, C3)])
        return carry
    lax.fori_loop(0, G3, chunk, None)


def _k_final(e1_ref, gs_ref, xr2_ref, xs2_ref, c2_ref, wde_ref, w2_ref, b2_ref,
             out_ref):
    h = jnp.maximum(_dot(e1_ref[...], wde_ref[...]) + gs_ref[...] + c2_ref[...], 0.0)
    o = _dot(h, w2_ref[...]) + b2_ref[...]
    rows = out_ref.shape[1]
    o1 = o[:, 0:1].reshape(1, rows, 128)
    o2 = o[:, 1:2].reshape(1, rows, 128)
    out_ref[...] = o1 * (xr2_ref[...] - o2 * xs2_ref[...])


def _cat(a, b):
    return jnp.concatenate([a, b], axis=1)


def _bd(a, b):
    z = jnp.zeros_like(a)
    return jnp.concatenate([jnp.concatenate([a, z], 1), jnp.concatenate([z, b], 1)], 0)


@jax.jit
def kernel(x, edge_attr, global_attr, params, edge_index):
    p1, p2 = params["w1"], params["w2"]
    u = global_attr

    We = _cat(p1["eb_W"][0:16], p2["eb_W"][0:16])
    Wr = _cat(p1["eb_W"][16:144], p2["eb_W"][16:144])
    Ws = _cat(p1["eb_W"][144:272], p2["eb_W"][144:272])
    Wu = _cat(p1["eb_W"][272:336], p2["eb_W"][272:336])
    eb_b = _cat(p1["eb_b"][None], p2["eb_b"][None])
    Wna = _bd(p1["nb_W"][0:64], p2["nb_W"][0:64])
    Wnx = _cat(p1["nb_W"][64:192], p2["nb_W"][64:192])
    Wnu = _cat(p1["nb_W"][192:256], p2["nb_W"][192:256])
    nb_b = _cat(p1["nb_b"][None], p2["nb_b"][None])
    gbW = jnp.stack([p1["gb_W"], p2["gb_W"]])
    gbb = jnp.stack([p1["gb_b"][None], p2["gb_b"][None]])
    Wde = _bd(p1["dec_W1"][0:64], p2["dec_W1"][0:64])
    Wdr = _bd(p1["dec_W1"][64:128], p2["dec_W1"][64:128])
    Wds = _bd(p1["dec_W1"][128:192], p2["dec_W1"][128:192])
    Wdu = jnp.stack([p1["dec_W1"][192:256], p2["dec_W1"][192:256]])
    db1 = jnp.stack([p1["dec_b1"][None], p2["dec_b1"][None]])
    z64 = jnp.zeros((64, 1), F32)
    W2 = jnp.concatenate([jnp.concatenate([p1["dec_W2"], z64], 0),
                          jnp.concatenate([z64, p2["dec_W2"]], 0),
                          jnp.zeros((128, 6), F32)], axis=1)
    b2 = jnp.concatenate([p1["dec_b2"], p2["dec_b2"], jnp.zeros((6,), F32)])[None]

    s_idx = edge_index[0]
    r_idx = edge_index[1]

    BN = 1000
    BE = 2000

    def full(shp):
        return pl.BlockSpec(shp, lambda *a: tuple(0 for _ in shp))

    tr_tab, ts_tab = pl.pallas_call(
        _k_tables,
        grid=(N // BN,),
        in_specs=[pl.BlockSpec((BN, DN), lambda i: (i, 0)), full((DN, H2)), full((DN, H2))],
        out_specs=[pl.BlockSpec((BN, H2), lambda i: (i, 0))] * 2,
        out_shape=[jax.ShapeDtypeStruct((N, H2), F32)] * 2,
    )(x, Wr, Ws)

    q = pl.pallas_call(
        _k_q,
        grid=(E // BE,),
        in_specs=[pl.BlockSpec((BE, 16), lambda i: (i, 0)), full((16, H2)),
                  full((1, 64)), full((64, H2)), full((1, H2))],
        out_specs=pl.BlockSpec((BE, H2), lambda i: (i, 0)),
        out_shape=jax.ShapeDtypeStruct((E, H2), F32),
    )(edge_attr, We, u, Wu, eb_b)

    sc1 = pl.kernel(
        _sc1_body,
        out_type=(jax.ShapeDtypeStruct((E, H2), F32),
                  jax.ShapeDtypeStruct((E,), F32),
                  jax.ShapeDtypeStruct((E,), F32),
                  jax.ShapeDtypeStruct((NC, NP, H2), F32)),
        mesh=_mesh,
        scratch_types=(
            pltpu.VMEM((C1,), jnp.int32),
            pltpu.VMEM((C1,), jnp.int32),
            pltpu.VMEM((C1, H2), F32),
            pltpu.VMEM((C1, H2), F32),
            pltpu.VMEM((C1, H2), F32),
            pltpu.VMEM((C1, H2), F32),
            pltpu.VMEM((C1,), F32),
            pltpu.VMEM((C1,), F32),
            pltpu.VMEM_SHARED((NP, H2), F32),
            pltpu.SemaphoreType.DMA,
            pltpu.SemaphoreType.DMA,
            pltpu.SemaphoreType.DMA,
            pltpu.SemaphoreType.DMA,
            pltpu.SemaphoreType.DMA,
            pltpu.SemaphoreType.DMA,
        ),
    )
    x2col = x[:, 2]
    e1, xr2, xs2, aggp = sc1(tr_tab, ts_tab, q, r_idx, s_idx, x2col)

    tbr, tbs, csa, csn = pl.pallas_call(
        _k_node,
        grid=(N // BN,),
        in_specs=[pl.BlockSpec((BN, H2), lambda i: (i, 0)),
                  pl.BlockSpec((BN, H2), lambda i: (i, 0)),
                  pl.BlockSpec((BN, DN), lambda i: (i, 0)),
                  full((1, 64)), full((DN, H2)), full((DN, H2)), full((64, H2)),
                  full((1, H2)), full((DN, H2)), full((DN, H2))],
        out_specs=[pl.BlockSpec((BN, H2), lambda i: (i, 0)),
                   pl.BlockSpec((BN, H2), lambda i: (i, 0)),
                   full((1, H2)), full((1, H2))],
        out_shape=[jax.ShapeDtypeStruct((N, H2), F32),
                   jax.ShapeDtypeStruct((N, H2), F32),
                   jax.ShapeDtypeStruct((1, H2), F32),
                   jax.ShapeDtypeStruct((1, H2), F32)],
    )(aggp[0, :N], aggp[1, :N], x, u, Wna, Wnx, Wnu, nb_b, Wdr, Wds)

    c2 = pl.pallas_call(
        _k_c2,
        in_specs=[full((1, H2)), full((1, H2)), full((1, 64)),
                  full((2, 192, 64)), full((2, 1, 64)), full((2, 64, 64)),
                  full((2, 1, 64))],
        out_specs=full((1, H2)),
        out_shape=jax.ShapeDtypeStruct((1, H2), F32),
    )(csa, csn, u, gbW, gbb, Wdu, db1)

    sc3 = pl.kernel(
        _sc3_body,
        out_type=jax.ShapeDtypeStruct((E, H2), F32),
        mesh=_mesh,
        scratch_types=(
            pltpu.VMEM((C3,), jnp.int32),
            pltpu.VMEM((C3,), jnp.int32),
            pltpu.VMEM((C3, H2), F32),
            pltpu.VMEM((C3, H2), F32),
            pltpu.SemaphoreType.DMA,
            pltpu.SemaphoreType.DMA,
            pltpu.SemaphoreType.DMA,
        ),
    )
    gsum = sc3(tbr, tbs, r_idx, s_idx)

    BF = 2560
    GF = E // BF
    BR = BF // 128
    outm = pl.pallas_call(
        _k_final,
        grid=(GF,),
        in_specs=[pl.BlockSpec((BF, H2), lambda i: (i, 0)),
                  pl.BlockSpec((BF, H2), lambda i: (i, 0)),
                  pl.BlockSpec((1, BR, 128), lambda i: (i, 0, 0)),
                  pl.BlockSpec((1, BR, 128), lambda i: (i, 0, 0)),
                  full((1, H2)), full((DN, H2)), full((DN, 8)), full((1, 8))],
        out_specs=pl.BlockSpec((1, BR, 128), lambda i: (i, 0, 0)),
        out_shape=jax.ShapeDtypeStruct((GF, BR, 128), F32),
    )(e1, gsum, xr2.reshape(GF, BR, 128), xs2.reshape(GF, BR, 128), c2, Wde, W2, b2)

    return outm.reshape(E, 1)


# speedup vs baseline: 1.3771x; 1.0922x over previous
import jax
import jax.numpy as jnp
from jax import lax
from jax.experimental import pallas as pl
from jax.experimental.pallas import tpu as pltpu, tpu_sc as plsc

N = 10000
E = 320000
DN = 128
H2 = 128

NC, NS = 2, 16
NW = NC * NS
C1 = 80
C3 = 200
NP = 10240
RPT = NP // NS

EA = 179200
EB = E - EA

_mesh = plsc.VectorSubcoreMesh(core_axis_name="c", subcore_axis_name="s")
F32 = jnp.float32


def _dot(a, b):
    return jnp.dot(a, b, preferred_element_type=F32)


def _k_tables(x_ref, wr_ref, ws_ref, tr_ref, ts_ref):
    xb = x_ref[...]
    tr_ref[...] = _dot(xb, wr_ref[...])
    ts_ref[...] = _dot(xb, ws_ref[...])


def _k_q(ea_ref, we_ref, u_ref, wu_ref, b_ref, q_ref):
    q_ref[...] = _dot(ea_ref[...], we_ref[...]) + _dot(u_ref[...], wu_ref[...]) + b_ref[...]


def _make_sc1(eoff, epw, g1):
    def body(tr_hbm, ts_hbm, q_hbm, r_hbm, s_hbm, x2_hbm,
             e1_hbm, xr2_hbm, xs2_hbm, agg_hbm,
             ir_v, is_v, tr_v, ts_v, q_v, e1_v, xr2_v, xs2_v, agg_sh,
             sem_r, sem_s, sem_q, sem_x1, sem_x2, sem_st):
        cid = lax.axis_index("c")
        sid = lax.axis_index("s")
        wid = sid * NC + cid

        zvec = jnp.zeros((16,), F32)

        def zfill(t, carry):
            e1_v[t // 8, pl.ds((t % 8) * 16, 16)] = zvec
            return carry
        lax.fori_loop(0, C1 * 8, zfill, None)

        def zcopy(k, carry):
            off = pl.multiple_of(sid * RPT + k * C1, C1)
            pltpu.sync_copy(e1_v, agg_sh.at[pl.ds(off, C1)])
            return carry
        lax.fori_loop(0, RPT // C1, zcopy, None)
        plsc.subcore_barrier()

        def chunk(g, carry):
            lbase = pl.multiple_of(wid * epw + g * C1, C1)
            gbase = pl.multiple_of(eoff + wid * epw + g * C1, C1)
            pltpu.sync_copy(r_hbm.at[pl.ds(gbase, C1)], ir_v)
            pltpu.sync_copy(s_hbm.at[pl.ds(gbase, C1)], is_v)
            cp_r = pltpu.async_copy(tr_hbm.at[ir_v], tr_v, sem_r)
            cp_s = pltpu.async_copy(ts_hbm.at[is_v], ts_v, sem_s)
            cp_q = pltpu.async_copy(q_hbm.at[pl.ds(lbase, C1)], q_v, sem_q)
            cp_x1 = pltpu.async_copy(x2_hbm.at[ir_v], xr2_v, sem_x1)
            cp_x2 = pltpu.async_copy(x2_hbm.at[is_v], xs2_v, sem_x2)
            cp_r.wait()
            cp_s.wait()
            cp_q.wait()

            @plsc.parallel_loop(0, C1, 1, unroll=4)
            def row(i):
                for j in range(H2 // 16):
                    sl = pl.ds(j * 16, 16)
                    v = q_v[i, sl] + tr_v[i, sl] + ts_v[i, sl]
                    e1_v[i, sl] = jnp.maximum(v, 0.0)

            cp_st = pltpu.async_copy(e1_v, e1_hbm.at[pl.ds(lbase, C1)], sem_st)
            pltpu.sync_copy(e1_v, agg_sh.at[ir_v], add=True)
            cp_x1.wait()
            cp_x2.wait()
            pltpu.sync_copy(xr2_v, xr2_hbm.at[pl.ds(lbase, C1)])
            pltpu.sync_copy(xs2_v, xs2_hbm.at[pl.ds(lbase, C1)])
            cp_st.wait()
            return carry
        lax.fori_loop(0, g1, chunk, None)

        plsc.subcore_barrier()
        doff = pl.multiple_of(sid * RPT, RPT)
        pltpu.sync_copy(agg_sh.at[pl.ds(doff, RPT)],
                        agg_hbm.at[cid, pl.ds(doff, RPT)])
    return body


def _k_node(a0_ref, a1_ref, a2_ref, a3_ref, x_ref, u_ref,
            wna_ref, wnx_ref, wnu_ref, nb_ref, wdr_ref, wds_ref,
            tbr_ref, tbs_ref, csa_ref, csn_ref):
    i = pl.program_id(0)
    agg = (a0_ref[...] + a1_ref[...]) + (a2_ref[...] + a3_ref[...])
    n1 = jnp.maximum(_dot(agg, wna_ref[...]) + _dot(x_ref[...], wnx_ref[...])
                     + _dot(u_ref[...], wnu_ref[...]) + nb_ref[...], 0.0)
    tbr_ref[...] = _dot(n1, wdr_ref[...])
    tbs_ref[...] = _dot(n1, wds_ref[...])

    @pl.when(i == 0)
    def _():
        csa_ref[...] = jnp.zeros_like(csa_ref)
        csn_ref[...] = jnp.zeros_like(csn_ref)
    csa_ref[...] += jnp.sum(agg, axis=0, keepdims=True)
    csn_ref[...] += jnp.sum(n1, axis=0, keepdims=True)


def _k_c2(csa_ref, csn_ref, u_ref, gbw_ref, gbb_ref, wdu_ref, db1_ref, c2_ref):
    u = u_ref[...]
    for t in range(2):
        sl = slice(t * 64, (t + 1) * 64)
        me1 = csa_ref[:, sl] * (1.0 / E)
        mn1 = csn_ref[:, sl] * (1.0 / N)
        g = jnp.concatenate([me1, mn1, u], axis=1)
        u1 = jnp.maximum(_dot(g, gbw_ref[t]) + gbb_ref[t], 0.0)
        c2_ref[:, sl] = db1_ref[t] + _dot(u1, wdu_ref[t])


def _make_sc3(eoff, epw, g3):
    def body(tbr_hbm, tbs_hbm, r_hbm, s_hbm, gsum_hbm,
             ir_v, is_v, tbr_v, tbs_v, sem_r, sem_s, sem_st):
        cid = lax.axis_index("c")
        sid = lax.axis_index("s")
        wid = sid * NC + cid

        def chunk(g, carry):
            lbase = pl.multiple_of(wid * epw + g * C3, C3)
            gbase = pl.multiple_of(eoff + wid * epw + g * C3, C3)
            pltpu.sync_copy(r_hbm.at[pl.ds(gbase, C3)], ir_v)
            pltpu.sync_copy(s_hbm.at[pl.ds(gbase, C3)], is_v)
            cp_r = pltpu.async_copy(tbr_hbm.at[ir_v], tbr_v, sem_r)
            cp_s = pltpu.async_copy(tbs_hbm.at[is_v], tbs_v, sem_s)
            cp_r.wait()
            cp_s.wait()

            @plsc.parallel_loop(0, C3, 1, unroll=4)
            def row(i):
                for j in range(H2 // 16):
                    sl = pl.ds(j * 16, 16)
                    tbr_v[i, sl] = tbr_v[i, sl] + tbs_v[i, sl]

            pltpu.sync_copy(tbr_v, gsum_hbm.at[pl.ds(lbase, C3)])
            return carry
        lax.fori_loop(0, g3, chunk, None)
    return body


def _k_final(e1_ref, gs_ref, xr2_ref, xs2_ref, c2_ref, wde_ref, w2_ref, b2_ref,
             out_ref):
    h = jnp.maximum(_dot(e1_ref[...], wde_ref[...]) + gs_ref[...] + c2_ref[...], 0.0)
    o = _dot(h, w2_ref[...]) + b2_ref[...]
    rows = out_ref.shape[1]
    o1 = o[:, 0:1].reshape(1, rows, 128)
    o2 = o[:, 1:2].reshape(1, rows, 128)
    out_ref[...] = o1 * (xr2_ref[...] - o2 * xs2_ref[...])


def _cat(a, b):
    return jnp.concatenate([a, b], axis=1)


def _bd(a, b):
    z = jnp.zeros_like(a)
    return jnp.concatenate([jnp.concatenate([a, z], 1), jnp.concatenate([z, b], 1)], 0)


def _sc1_call(eoff, esz):
    epw = esz // NW
    return pl.kernel(
        _make_sc1(eoff, epw, epw // C1),
        out_type=(jax.ShapeDtypeStruct((esz, H2), F32),
                  jax.ShapeDtypeStruct((esz,), F32),
                  jax.ShapeDtypeStruct((esz,), F32),
                  jax.ShapeDtypeStruct((NC, NP, H2), F32)),
        mesh=_mesh,
        scratch_types=(
            pltpu.VMEM((C1,), jnp.int32),
            pltpu.VMEM((C1,), jnp.int32),
            pltpu.VMEM((C1, H2), F32),
            pltpu.VMEM((C1, H2), F32),
            pltpu.VMEM((C1, H2), F32),
            pltpu.VMEM((C1, H2), F32),
            pltpu.VMEM((C1,), F32),
            pltpu.VMEM((C1,), F32),
            pltpu.VMEM_SHARED((NP, H2), F32),
            pltpu.SemaphoreType.DMA,
            pltpu.SemaphoreType.DMA,
            pltpu.SemaphoreType.DMA,
            pltpu.SemaphoreType.DMA,
            pltpu.SemaphoreType.DMA,
            pltpu.SemaphoreType.DMA,
        ),
    )


def _sc3_call(eoff, esz):
    epw = esz // NW
    return pl.kernel(
        _make_sc3(eoff, epw, epw // C3),
        out_type=jax.ShapeDtypeStruct((esz, H2), F32),
        mesh=_mesh,
        scratch_types=(
            pltpu.VMEM((C3,), jnp.int32),
            pltpu.VMEM((C3,), jnp.int32),
            pltpu.VMEM((C3, H2), F32),
            pltpu.VMEM((C3, H2), F32),
            pltpu.SemaphoreType.DMA,
            pltpu.SemaphoreType.DMA,
            pltpu.SemaphoreType.DMA,
        ),
    )


@jax.jit
def kernel(x, edge_attr, global_attr, params, edge_index):
    p1, p2 = params["w1"], params["w2"]
    u = global_attr

    We = _cat(p1["eb_W"][0:16], p2["eb_W"][0:16])
    Wr = _cat(p1["eb_W"][16:144], p2["eb_W"][16:144])
    Ws = _cat(p1["eb_W"][144:272], p2["eb_W"][144:272])
    Wu = _cat(p1["eb_W"][272:336], p2["eb_W"][272:336])
    eb_b = _cat(p1["eb_b"][None], p2["eb_b"][None])
    Wna = _bd(p1["nb_W"][0:64], p2["nb_W"][0:64])
    Wnx = _cat(p1["nb_W"][64:192], p2["nb_W"][64:192])
    Wnu = _cat(p1["nb_W"][192:256], p2["nb_W"][192:256])
    nb_b = _cat(p1["nb_b"][None], p2["nb_b"][None])
    gbW = jnp.stack([p1["gb_W"], p2["gb_W"]])
    gbb = jnp.stack([p1["gb_b"][None], p2["gb_b"][None]])
    Wde = _bd(p1["dec_W1"][0:64], p2["dec_W1"][0:64])
    Wdr = _bd(p1["dec_W1"][64:128], p2["dec_W1"][64:128])
    Wds = _bd(p1["dec_W1"][128:192], p2["dec_W1"][128:192])
    Wdu = jnp.stack([p1["dec_W1"][192:256], p2["dec_W1"][192:256]])
    db1 = jnp.stack([p1["dec_b1"][None], p2["dec_b1"][None]])
    z64 = jnp.zeros((64, 1), F32)
    W2 = jnp.concatenate([jnp.concatenate([p1["dec_W2"], z64], 0),
                          jnp.concatenate([z64, p2["dec_W2"]], 0),
                          jnp.zeros((128, 6), F32)], axis=1)
    b2 = jnp.concatenate([p1["dec_b2"], p2["dec_b2"], jnp.zeros((6,), F32)])[None]

    s_idx = edge_index[0]
    r_idx = edge_index[1]
    x2col = x[:, 2]

    BN = 1000
    BE = 2560

    def full(shp):
        return pl.BlockSpec(shp, lambda *a: tuple(0 for _ in shp))

    tr_tab, ts_tab = pl.pallas_call(
        _k_tables,
        grid=(N // BN,),
        in_specs=[pl.BlockSpec((BN, DN), lambda i: (i, 0)), full((DN, H2)), full((DN, H2))],
        out_specs=[pl.BlockSpec((BN, H2), lambda i: (i, 0))] * 2,
        out_shape=[jax.ShapeDtypeStruct((N, H2), F32)] * 2,
    )(x, Wr, Ws)

    def q_half(eoff, esz):
        ob = eoff // BE
        return pl.pallas_call(
            _k_q,
            grid=(esz // BE,),
            in_specs=[pl.BlockSpec((BE, 16), lambda i: (i + ob, 0)), full((16, H2)),
                      full((1, 64)), full((64, H2)), full((1, H2))],
            out_specs=pl.BlockSpec((BE, H2), lambda i: (i, 0)),
            out_shape=jax.ShapeDtypeStruct((esz, H2), F32),
        )(edge_attr, We, u, Wu, eb_b)

    qA = q_half(0, EA)
    qB = q_half(EA, EB)

    e1A, xr2A, xs2A, aggA = _sc1_call(0, EA)(tr_tab, ts_tab, qA, r_idx, s_idx, x2col)
    e1B, xr2B, xs2B, aggB = _sc1_call(EA, EB)(tr_tab, ts_tab, qB, r_idx, s_idx, x2col)

    tbr, tbs, csa, csn = pl.pallas_call(
        _k_node,
        grid=(N // BN,),
        in_specs=[pl.BlockSpec((BN, H2), lambda i: (i, 0)),
                  pl.BlockSpec((BN, H2), lambda i: (i, 0)),
                  pl.BlockSpec((BN, H2), lambda i: (i, 0)),
                  pl.BlockSpec((BN, H2), lambda i: (i, 0)),
                  pl.BlockSpec((BN, DN), lambda i: (i, 0)),
                  full((1, 64)), full((DN, H2)), full((DN, H2)), full((64, H2)),
                  full((1, H2)), full((DN, H2)), full((DN, H2))],
        out_specs=[pl.BlockSpec((BN, H2), lambda i: (i, 0)),
                   pl.BlockSpec((BN, H2), lambda i: (i, 0)),
                   full((1, H2)), full((1, H2))],
        out_shape=[jax.ShapeDtypeStruct((N, H2), F32),
                   jax.ShapeDtypeStruct((N, H2), F32),
                   jax.ShapeDtypeStruct((1, H2), F32),
                   jax.ShapeDtypeStruct((1, H2), F32)],
    )(aggA[0, :N], aggA[1, :N], aggB[0, :N], aggB[1, :N], x, u,
      Wna, Wnx, Wnu, nb_b, Wdr, Wds)

    c2 = pl.pallas_call(
        _k_c2,
        in_specs=[full((1, H2)), full((1, H2)), full((1, 64)),
                  full((2, 192, 64)), full((2, 1, 64)), full((2, 64, 64)),
                  full((2, 1, 64))],
        out_specs=full((1, H2)),
        out_shape=jax.ShapeDtypeStruct((1, H2), F32),
    )(csa, csn, u, gbW, gbb, Wdu, db1)

    gsumA = _sc3_call(0, EA)(tbr, tbs, r_idx, s_idx)
    gsumB = _sc3_call(EA, EB)(tbr, tbs, r_idx, s_idx)

    BR = BE // 128

    def final_half(e1, gsum, xr2, xs2, esz):
        gf = esz // BE
        return pl.pallas_call(
            _k_final,
            grid=(gf,),
            in_specs=[pl.BlockSpec((BE, H2), lambda i: (i, 0)),
                      pl.BlockSpec((BE, H2), lambda i: (i, 0)),
                      pl.BlockSpec((1, BR, 128), lambda i: (i, 0, 0)),
                      pl.BlockSpec((1, BR, 128), lambda i: (i, 0, 0)),
                      full((1, H2)), full((DN, H2)), full((DN, 8)), full((1, 8))],
            out_specs=pl.BlockSpec((1, BR, 128), lambda i: (i, 0, 0)),
            out_shape=jax.ShapeDtypeStruct((gf, BR, 128), F32),
        )(e1, gsum, xr2.reshape(gf, BR, 128), xs2.reshape(gf, BR, 128),
          c2, Wde, W2, b2)

    outA = final_half(e1A, gsumA, xr2A, xs2A, EA)
    outB = final_half(e1B, gsumB, xr2B, xs2B, EB)

    return jnp.concatenate([outA.reshape(EA, 1), outB.reshape(EB, 1)], axis=0)
